# baseline (speedup 1.0000x reference)
import functools

import jax
import jax.numpy as jnp
from jax import lax
from jax.experimental import pallas as pl
from jax.experimental.pallas import tpu as pltpu
from jax.experimental.pallas import tpu_sc as plsc

N = 10000
E = 160000
F = 128
FD = 128
CHUNK = 80
NCHUNKS = E // CHUNK
CPAD = 2048
NB = 1000
NBLK = N // NB
WB = 624
AROWS = N + 8
ZPW = 624

_mesh = lambda: plsc.VectorSubcoreMesh(
    core_axis_name="c", subcore_axis_name="s", num_cores=2, num_subcores=16)


def _zero_acc(zeros_hbm, acc, s):
    @pl.when(s < 15)
    def _():
        pltpu.sync_copy(zeros_hbm.at[pl.ds(0, ZPW)],
                        acc.at[pl.ds(s * ZPW, ZPW)])

    @pl.when(s == 15)
    def _():
        pltpu.sync_copy(zeros_hbm.at[pl.ds(0, AROWS - 15 * ZPW)],
                        acc.at[pl.ds(15 * ZPW, AROWS - 15 * ZPW)])


def _writeback_n(acc, out2d, s):
    @pl.when(s < 15)
    def _():
        pltpu.sync_copy(acc.at[pl.ds(s * WB, WB)], out2d.at[pl.ds(s * WB, WB)])

    @pl.when(s == 15)
    def _():
        pltpu.sync_copy(acc.at[pl.ds(15 * WB, N - 15 * WB)],
                        out2d.at[pl.ds(15 * WB, N - 15 * WB)])


def _make_prop(split_edges):
    nrows = 64 if split_edges else 128
    nbuf = 4
    grp = 32
    ngroups = nrows // grp

    @functools.partial(
        pl.kernel,
        out_type=jax.ShapeDtypeStruct((2, N, F), jnp.float32),
        mesh=_mesh(),
        scratch_types=(
            [pltpu.VMEM((grp, CHUNK), jnp.int32),
             pltpu.VMEM((grp, CHUNK), jnp.int32)]
            + [pltpu.VMEM((CHUNK, F), jnp.float32) for _ in range(nbuf)]
            + [pltpu.VMEM_SHARED((AROWS, F), jnp.float32)]
            + [pltpu.SemaphoreType.DMA for _ in range(nbuf)]
        ),
    )
    def prop(src2_hbm, dst2_hbm, xs_hbm, zeros_hbm, out_hbm, *refs):
        gi_blk, di_blk = refs[0], refs[1]
        rows = refs[2:2 + nbuf]
        acc = refs[2 + nbuf]
        sems = refs[3 + nbuf:]
        c = lax.axis_index("c")
        s = lax.axis_index("s")
        _zero_acc(zeros_hbm, acc, s)
        if split_edges:
            base = (c * 16 + s) * nrows
            gsel = 0
        else:
            base = s * nrows
            gsel = c
        plsc.subcore_barrier()

        def start_gather(j, k):
            pltpu.async_copy(xs_hbm.at[gi_blk.at[j]], rows[k], sems[k])

        def finish_chunk(j, k):
            pltpu.make_async_copy(xs_hbm.at[gi_blk.at[j]], rows[k],
                                  sems[k]).wait()
            pltpu.sync_copy(rows[k], acc.at[di_blk.at[j]], add=True)

        def group_body(g, carry):
            pltpu.sync_copy(src2_hbm.at[gsel, pl.ds(base + g * grp, grp)],
                            gi_blk)
            pltpu.sync_copy(dst2_hbm.at[pl.ds(base + g * grp, grp)], di_blk)
            for k in range(nbuf):
                start_gather(k, k)

            def body(i, carry2):
                for k in range(nbuf):
                    j = i * nbuf + k
                    finish_chunk(j, k)
                    start_gather(j + nbuf, k)
                return carry2

            lax.fori_loop(0, grp // nbuf - 1, body, 0)
            for k in range(nbuf):
                finish_chunk(grp - nbuf + k, k)
            return carry

        lax.fori_loop(0, ngroups, group_body, 0)

        plsc.subcore_barrier()
        _writeback_n(acc, out_hbm.at[c], s)

    return prop


_prop2 = _make_prop(split_edges=False)
_prop1 = _make_prop(split_edges=True)

_DEG_ROWS = 64


@functools.partial(
    pl.kernel,
    out_type=jax.ShapeDtypeStruct((2, N, FD), jnp.float32),
    mesh=_mesh(),
    scratch_types=[
        pltpu.VMEM((_DEG_ROWS, CHUNK), jnp.int32),
        pltpu.VMEM((CHUNK, FD), jnp.float32),
        pltpu.VMEM_SHARED((AROWS, FD), jnp.float32),
        pltpu.SemaphoreType.DMA,
    ],
)
def _deg_kernel(dst2_hbm, ones_hbm, zeros_hbm, out_hbm, di_blk, ones_v, acc,
                sem):
    c = lax.axis_index("c")
    s = lax.axis_index("s")
    _zero_acc(zeros_hbm, acc, s)
    wid = c * 16 + s
    pltpu.sync_copy(dst2_hbm.at[pl.ds(wid * _DEG_ROWS, _DEG_ROWS)], di_blk)
    pltpu.sync_copy(ones_hbm, ones_v)
    plsc.subcore_barrier()

    def body(i, carry):
        pltpu.sync_copy(ones_v, acc.at[di_blk.at[i]], add=True)
        return carry

    lax.fori_loop(0, _DEG_ROWS, body, 0)
    plsc.subcore_barrier()
    _writeback_n(acc, out_hbm.at[c], s)


def _tc_pre_body(x_ref, degp_ref, xs0_ref, dinv16_ref):
    deg = degp_ref[0, :, 0] + degp_ref[1, :, 0] + 1.0
    dinv = lax.rsqrt(deg)
    dinv16_ref[...] = jnp.broadcast_to(dinv[:, None], (NB, 16))
    xs = x_ref[...] * dinv[:, None]
    for j in range(2):
        xs0_ref[j] = xs[:, F * j:F * (j + 1)]


def _tc1_body(s0_ref, xs0_ref, dinv16_ref, w1_ref, b1_ref, wm_ref,
              xs1a_ref, xs1b_ref):
    dinv = dinv16_ref[:, 0:1]
    z0 = jnp.concatenate(
        [s0_ref[j] + xs0_ref[j] for j in range(2)], axis=1) * dinv
    h = jnp.maximum(
        jnp.dot(z0, w1_ref[...], preferred_element_type=jnp.float32)
        + b1_ref[...], 0.0)
    m = jnp.dot(h, wm_ref[...], preferred_element_type=jnp.float32)
    xs1 = m * dinv
    for j in range(2):
        xs1a_ref[j] = xs1[:, F * j:F * (j + 1)]
        xs1b_ref[j] = xs1[:, 256 + F * j:256 + F * (j + 1)]


def _tc2_body(s1a_ref, s1b_ref, xs1a_ref, xs1b_ref, dinv16_ref, bm_ref,
              w2p_ref, xs2_ref):
    dinv = dinv16_ref[:, 0:1]
    scat = jnp.concatenate(
        [s1a_ref[j] for j in range(2)] + [s1b_ref[j] for j in range(2)],
        axis=1)
    xcat = jnp.concatenate(
        [xs1a_ref[j] for j in range(2)] + [xs1b_ref[j] for j in range(2)],
        axis=1)
    z1 = (scat + xcat) * dinv + bm_ref[...]
    h2 = jnp.maximum(z1, 0.0)
    c2 = jnp.dot(h2, w2p_ref[...], preferred_element_type=jnp.float32)
    xs2_ref[...] = c2 * dinv


def _tc3_body(s2p_ref, xs2_ref, dinv16_ref, b2_ref, out_ref):
    dinv = dinv16_ref[:, 0:1]
    z = (s2p_ref[0] + s2p_ref[1] + xs2_ref[...]) * dinv
    out_ref[...] = z[:, :64] + b2_ref[...]


def _row3(d0, d2):
    return pl.BlockSpec((d0, NB, d2), lambda i: (0, i, 0))


def _row2(d1):
    return pl.BlockSpec((NB, d1), lambda i: (i, 0))


def _full(shape):
    return pl.BlockSpec(shape, lambda i: tuple(0 for _ in shape))


def kernel(x, adj, W1, b1, Wm, bm, W2, b2):
    src32 = adj[0].astype(jnp.int32)
    dst32 = adj[1].astype(jnp.int32)
    npad = CPAD * CHUNK - E
    srcp = jnp.concatenate([src32, jnp.zeros((npad,), jnp.int32)])
    src2 = jnp.stack([srcp, srcp + N]).reshape(2, CPAD, CHUNK)
    dst2 = jnp.concatenate(
        [dst32, jnp.full((npad,), N, jnp.int32)]).reshape(CPAD, CHUNK)
    zerosF = jnp.zeros((AROWS - 15 * ZPW, F), jnp.float32)
    onesD = jnp.ones((CHUNK, FD), jnp.float32)
    W2p = jnp.pad(W2, ((0, 0), (0, F - 64)))

    degp = _deg_kernel(dst2, onesD, zerosF)

    xs0, dinv16 = pl.pallas_call(
        _tc_pre_body,
        grid=(NBLK,),
        in_specs=[_row2(256), _row3(2, FD)],
        out_specs=[_row3(2, F), _row2(16)],
        out_shape=[jax.ShapeDtypeStruct((2, N, F), jnp.float32),
                   jax.ShapeDtypeStruct((N, 16), jnp.float32)],
    )(x, degp)

    s0 = _prop2(src2, dst2, xs0.reshape(2 * N, F), zerosF)

    xs1a, xs1b = pl.pallas_call(
        _tc1_body,
        grid=(NBLK,),
        in_specs=[_row3(2, F), _row3(2, F), _row2(16), _full((256, 512)),
                  _full((1, 512)), _full((512, 512))],
        out_specs=[_row3(2, F), _row3(2, F)],
        out_shape=[jax.ShapeDtypeStruct((2, N, F), jnp.float32),
                   jax.ShapeDtypeStruct((2, N, F), jnp.float32)],
    )(s0, xs0, dinv16, W1, b1.reshape(1, 512), Wm)

    s1a = _prop2(src2, dst2, xs1a.reshape(2 * N, F), zerosF)
    s1b = _prop2(src2, dst2, xs1b.reshape(2 * N, F), zerosF)

    xs2 = pl.pallas_call(
        _tc2_body,
        grid=(NBLK,),
        in_specs=[_row3(2, F), _row3(2, F), _row3(2, F), _row3(2, F),
                  _row2(16), _full((1, 512)), _full((512, F))],
        out_specs=_row2(F),
        out_shape=jax.ShapeDtypeStruct((N, F), jnp.float32),
    )(s1a, s1b, xs1a, xs1b, dinv16, bm.reshape(1, 512), W2p)

    s2p = _prop1(src2, dst2, xs2, zerosF)

    out = pl.pallas_call(
        _tc3_body,
        grid=(NBLK,),
        in_specs=[_row3(2, F), _row2(F), _row2(16), _full((1, 64))],
        out_specs=_row2(64),
        out_shape=jax.ShapeDtypeStruct((N, 64), jnp.float32),
    )(s2p, xs2, dinv16, b2.reshape(1, 64))

    return out
